# baseline (speedup 1.0000x reference)
import jax
import jax.numpy as jnp
from jax import lax
from jax.experimental import pallas as pl
from jax.experimental.pallas import tpu as pltpu
from jax.experimental.pallas import tpu_sc as plsc

N_NODES = 10000
N_EDGES = 320000
D = 128

NC = 2
NS = 16
NW = NC * NS

CHUNK = 128
B = 8
NB = 10
K = NB * B
E_PER_W = K * CHUNK
EPAD = NW * E_PER_W
NPAD = 10240
ROWS_PER_TILE = NPAD // NS

_MESH = plsc.VectorSubcoreMesh(core_axis_name="c", subcore_axis_name="s",
                               num_cores=NC, num_subcores=NS)
_SC_PARAMS = pltpu.CompilerParams(needs_layout_passes=False)


def _deg_body(src_hbm, dst_hbm, dego_hbm, degi_hbm, sidx, didx, ho, hi, sem):
    cid = lax.axis_index("c")
    sid = lax.axis_index("s")
    wid = sid * NC + cid

    pltpu.async_copy(src_hbm.at[wid], sidx, sem).wait()
    pltpu.async_copy(dst_hbm.at[wid], didx, sem).wait()

    zeros16 = jnp.zeros((16,), jnp.float32)

    def zero_body(i, _):
        ho[pl.ds(i * 16, 16)] = zeros16
        hi[pl.ds(i * 16, 16)] = zeros16
        return _

    lax.fori_loop(0, NPAD // 16, zero_body, None)

    ones16 = jnp.ones((16,), jnp.float32)

    def edge_body(i, _):
        s = sidx[pl.ds(i * 16, 16)]
        d = didx[pl.ds(i * 16, 16)]
        plsc.addupdate_scatter(ho, [s], ones16)
        plsc.addupdate_scatter(hi, [d], ones16)
        return _

    lax.fori_loop(0, E_PER_W // 16, edge_body, None)

    pltpu.sync_copy(ho, dego_hbm.at[wid])
    pltpu.sync_copy(hi, degi_hbm.at[wid])


_deg_kernel = pl.kernel(
    _deg_body,
    out_type=(jax.ShapeDtypeStruct((NW, NPAD), jnp.float32),
              jax.ShapeDtypeStruct((NW, NPAD), jnp.float32)),
    mesh=_MESH,
    scratch_types=[
        pltpu.VMEM((E_PER_W,), jnp.int32),
        pltpu.VMEM((E_PER_W,), jnp.int32),
        pltpu.VMEM((NPAD,), jnp.float32),
        pltpu.VMEM((NPAD,), jnp.float32),
        pltpu.SemaphoreType.DMA,
    ],
    compiler_params=_SC_PARAMS,
)


def _scale_body(feats_ref, degp_ref, h_ref):
    deg = jnp.sum(degp_ref[:, 0:N_NODES], axis=0)
    norm = jnp.where(deg > 0, lax.rsqrt(deg), 0.0)
    h_ref[...] = feats_ref[...] * norm[:, None]


def _scale(feats, dego_p):
    return pl.pallas_call(
        _scale_body,
        out_shape=jax.ShapeDtypeStruct((N_NODES, D), jnp.float32),
    )(feats, dego_p)


def _agg_body(h_hbm, src_hbm, dst_hbm, out0_hbm, out1_hbm,
              sbufA, dbufA, sbufB, dbufB, rows0, rows1, acc,
              semA, semB, semIA, semIB):
    cid = lax.axis_index("c")
    sid = lax.axis_index("s")
    wid = sid * NC + cid

    pltpu.async_copy(src_hbm.at[wid, 0], sbufA, semIA)
    pltpu.async_copy(dst_hbm.at[wid, 0], dbufA, semIA)
    pltpu.async_copy(src_hbm.at[wid, 1], sbufB, semIB)
    pltpu.async_copy(dst_hbm.at[wid, 1], dbufB, semIB)

    zeros16 = jnp.zeros((16,), jnp.float32)

    def zero_body(r, _):
        for c in range(D // 16):
            rows0[r, pl.ds(c * 16, 16)] = zeros16
        return _

    lax.fori_loop(0, CHUNK, zero_body, None)
    for k in range(ROWS_PER_TILE // CHUNK):
        pltpu.sync_copy(rows0, acc.at[pl.ds(sid * ROWS_PER_TILE + k * CHUNK, CHUNK)])
    plsc.subcore_barrier()

    rows = (rows0, rows1)
    sems = (semA, semB)

    def batch(bi, sbuf, dbuf, semI):
        pltpu.make_async_copy(src_hbm.at[wid, 0], sbuf, semI).wait()
        pltpu.make_async_copy(dst_hbm.at[wid, 0], dbuf, semI).wait()
        pltpu.async_copy(h_hbm.at[sbuf.at[0]], rows0, semA)
        for i in range(1, B):
            p, q = i & 1, (i - 1) & 1
            pltpu.async_copy(h_hbm.at[sbuf.at[i]], rows[p], sems[p])
            pltpu.make_async_copy(h_hbm.at[sbuf.at[i - 1]], rows[q], sems[q]).wait()
            pltpu.sync_copy(rows[q], acc.at[dbuf.at[i - 1]], add=True)
        pltpu.make_async_copy(h_hbm.at[sbuf.at[B - 1]], rows[(B - 1) & 1],
                              sems[(B - 1) & 1]).wait()
        pltpu.sync_copy(rows[(B - 1) & 1], acc.at[dbuf.at[B - 1]], add=True)
        @pl.when(bi + 2 < NB)
        def _():
            pltpu.async_copy(src_hbm.at[wid, bi + 2], sbuf, semI)
            pltpu.async_copy(dst_hbm.at[wid, bi + 2], dbuf, semI)

    def pair_body(t, _):
        batch(2 * t, sbufA, dbufA, semIA)
        batch(2 * t + 1, sbufB, dbufB, semIB)
        return _

    lax.fori_loop(0, NB // 2, pair_body, None)

    plsc.subcore_barrier()
    sl = pl.ds(sid * ROWS_PER_TILE, ROWS_PER_TILE)

    @pl.when(cid == 0)
    def _():
        pltpu.sync_copy(acc.at[sl], out0_hbm.at[sl])

    @pl.when(cid == 1)
    def _():
        pltpu.sync_copy(acc.at[sl], out1_hbm.at[sl])


_agg_kernel = pl.kernel(
    _agg_body,
    out_type=(jax.ShapeDtypeStruct((NPAD, D), jnp.float32),
              jax.ShapeDtypeStruct((NPAD, D), jnp.float32)),
    mesh=_MESH,
    scratch_types=[
        pltpu.VMEM((B, CHUNK), jnp.int32),
        pltpu.VMEM((B, CHUNK), jnp.int32),
        pltpu.VMEM((B, CHUNK), jnp.int32),
        pltpu.VMEM((B, CHUNK), jnp.int32),
        pltpu.VMEM((CHUNK, D), jnp.float32),
        pltpu.VMEM((CHUNK, D), jnp.float32),
        pltpu.VMEM_SHARED((NPAD, D), jnp.float32),
        pltpu.SemaphoreType.DMA,
        pltpu.SemaphoreType.DMA,
        pltpu.SemaphoreType.DMA,
        pltpu.SemaphoreType.DMA,
    ],
    compiler_params=_SC_PARAMS,
)


def _out_body(a0_ref, a1_ref, degp_ref, w_ref, b_ref, pa_ref, o_ref):
    deg = jnp.sum(degp_ref[...], axis=0)
    norm = jnp.where(deg > 0, lax.rsqrt(deg), 0.0)
    rst = (a0_ref[0:N_NODES] + a1_ref[0:N_NODES]) * norm[0:N_NODES, None]
    o = jnp.dot(rst, w_ref[...], preferred_element_type=jnp.float32) + b_ref[...]
    a = pa_ref[0, 0]
    o_ref[...] = jnp.where(o >= 0, o, a * o)


def _finish(a0, a1, degi_p, W, b2, pa2):
    return pl.pallas_call(
        _out_body,
        in_specs=[
            pl.BlockSpec(memory_space=pltpu.VMEM),
            pl.BlockSpec(memory_space=pltpu.VMEM),
            pl.BlockSpec(memory_space=pltpu.VMEM),
            pl.BlockSpec(memory_space=pltpu.VMEM),
            pl.BlockSpec(memory_space=pltpu.VMEM),
            pl.BlockSpec(memory_space=pltpu.SMEM),
        ],
        out_shape=jax.ShapeDtypeStruct((N_NODES, D), jnp.float32),
    )(a0, a1, degi_p, W, b2, pa2)


def kernel(feats, edge_index, W, b, prelu_a):
    src = edge_index[0].astype(jnp.int32)
    dst = edge_index[1].astype(jnp.int32)
    pad = N_NODES + (jnp.arange(EPAD - N_EDGES, dtype=jnp.int32) % (NPAD - N_NODES))
    pad0 = jnp.zeros((EPAD - N_EDGES,), jnp.int32)
    src_p = jnp.concatenate([src, pad]).reshape(NW, E_PER_W)
    dst_p = jnp.concatenate([dst, pad]).reshape(NW, E_PER_W)
    src_c = jnp.concatenate([src, pad0]).reshape(NW, NB, B, CHUNK)
    dst_c = dst_p.reshape(NW, NB, B, CHUNK)

    dego_p, degi_p = _deg_kernel(src_p, dst_p)
    h = _scale(feats, dego_p)
    a0, a1 = _agg_kernel(h, src_c, dst_c)
    out = _finish(a0, a1, degi_p, W,
                  b.reshape(1, D), prelu_a.reshape(1, 1))
    return out


# speedup vs baseline: 3.1470x; 3.1470x over previous
import jax
import jax.numpy as jnp
from jax import lax
from jax.experimental import pallas as pl
from jax.experimental.pallas import tpu as pltpu
from jax.experimental.pallas import tpu_sc as plsc

N_NODES = 10000
N_EDGES = 320000
D = 128

NC = 2
NS = 16
NW = NC * NS

CHUNK = 128
B = 8
NB = 10
K = NB * B
E_PER_W = K * CHUNK
EPAD = NW * E_PER_W
NPAD = 10240
ROWS_PER_TILE = NPAD // NS

_MESH = plsc.VectorSubcoreMesh(core_axis_name="c", subcore_axis_name="s",
                               num_cores=NC, num_subcores=NS)
_SC_PARAMS = pltpu.CompilerParams(needs_layout_passes=False)


def _deg_body(src_hbm, dst_hbm, dego_hbm, degi_hbm, sidx, didx, ho, hi, sem):
    cid = lax.axis_index("c")
    sid = lax.axis_index("s")
    wid = sid * NC + cid

    pltpu.async_copy(src_hbm.at[wid], sidx, sem).wait()
    pltpu.async_copy(dst_hbm.at[wid], didx, sem).wait()

    zeros16 = jnp.zeros((16,), jnp.float32)

    def zero_body(i, _):
        ho[pl.ds(i * 16, 16)] = zeros16
        hi[pl.ds(i * 16, 16)] = zeros16
        return _

    lax.fori_loop(0, NPAD // 16, zero_body, None)

    ones16 = jnp.ones((16,), jnp.float32)

    def edge_body(i, _):
        s = sidx[pl.ds(i * 16, 16)]
        d = didx[pl.ds(i * 16, 16)]
        plsc.addupdate_scatter(ho, [s], ones16)
        plsc.addupdate_scatter(hi, [d], ones16)
        return _

    lax.fori_loop(0, E_PER_W // 16, edge_body, None)

    pltpu.sync_copy(ho, dego_hbm.at[wid])
    pltpu.sync_copy(hi, degi_hbm.at[wid])


_deg_kernel = pl.kernel(
    _deg_body,
    out_type=(jax.ShapeDtypeStruct((NW, NPAD), jnp.float32),
              jax.ShapeDtypeStruct((NW, NPAD), jnp.float32)),
    mesh=_MESH,
    scratch_types=[
        pltpu.VMEM((E_PER_W,), jnp.int32),
        pltpu.VMEM((E_PER_W,), jnp.int32),
        pltpu.VMEM((NPAD,), jnp.float32),
        pltpu.VMEM((NPAD,), jnp.float32),
        pltpu.SemaphoreType.DMA,
    ],
    compiler_params=_SC_PARAMS,
)


def _scale_body(feats_ref, degp_ref, h_ref):
    deg = jnp.sum(degp_ref[:, 0:N_NODES], axis=0)
    norm = jnp.where(deg > 0, lax.rsqrt(deg), 0.0)
    h_ref[...] = feats_ref[...] * norm[:, None]


def _scale(feats, dego_p):
    return pl.pallas_call(
        _scale_body,
        out_shape=jax.ShapeDtypeStruct((N_NODES, D), jnp.float32),
    )(feats, dego_p)


def _agg_body(h_hbm, src_hbm, dst_hbm, out0_hbm, out1_hbm,
              sbufA, dbufA, sbufB, dbufB, rows0, rows1, acc,
              semA, semB, semIA, semIB):
    cid = lax.axis_index("c")
    sid = lax.axis_index("s")
    wid = sid * NC + cid

    pltpu.async_copy(src_hbm.at[wid, 0], sbufA, semIA)
    pltpu.async_copy(dst_hbm.at[wid, 0], dbufA, semIA)
    pltpu.async_copy(src_hbm.at[wid, 1], sbufB, semIB)
    pltpu.async_copy(dst_hbm.at[wid, 1], dbufB, semIB)

    zeros16 = jnp.zeros((16,), jnp.float32)

    def zero_body(r, _):
        for c in range(D // 16):
            rows0[r, pl.ds(c * 16, 16)] = zeros16
        return _

    lax.fori_loop(0, CHUNK, zero_body, None)
    for k in range(ROWS_PER_TILE // CHUNK):
        pltpu.sync_copy(rows0, acc.at[pl.ds(sid * ROWS_PER_TILE + k * CHUNK, CHUNK)])
    plsc.subcore_barrier()

    rows = (rows0, rows1)
    sems = (semA, semB)

    def batch(bi, sbuf, dbuf, semI):
        pltpu.make_async_copy(src_hbm.at[wid, 0], sbuf, semI).wait()
        pltpu.make_async_copy(dst_hbm.at[wid, 0], dbuf, semI).wait()
        pltpu.async_copy(h_hbm.at[sbuf.at[0]], rows0, semA)
        for i in range(1, B):
            p, q = i & 1, (i - 1) & 1
            pltpu.async_copy(h_hbm.at[sbuf.at[i]], rows[p], sems[p])
            pltpu.make_async_copy(h_hbm.at[sbuf.at[i - 1]], rows[q], sems[q]).wait()
            pltpu.sync_copy(rows[q], acc.at[dbuf.at[i - 1]], add=True)
        pltpu.make_async_copy(h_hbm.at[sbuf.at[B - 1]], rows[(B - 1) & 1],
                              sems[(B - 1) & 1]).wait()
        pltpu.sync_copy(rows[(B - 1) & 1], acc.at[dbuf.at[B - 1]], add=True)
        @pl.when(bi + 2 < NB)
        def _():
            pltpu.async_copy(src_hbm.at[wid, bi + 2], sbuf, semI)
            pltpu.async_copy(dst_hbm.at[wid, bi + 2], dbuf, semI)

    def pair_body(t, _):
        batch(2 * t, sbufA, dbufA, semIA)
        batch(2 * t + 1, sbufB, dbufB, semIB)
        return _

    lax.fori_loop(0, NB // 2, pair_body, None)

    plsc.subcore_barrier()
    sl = pl.ds(sid * ROWS_PER_TILE, ROWS_PER_TILE)

    @pl.when(cid == 0)
    def _():
        pltpu.sync_copy(acc.at[sl], out0_hbm.at[sl])

    @pl.when(cid == 1)
    def _():
        pltpu.sync_copy(acc.at[sl], out1_hbm.at[sl])


_agg_kernel = pl.kernel(
    _agg_body,
    out_type=(jax.ShapeDtypeStruct((NPAD, D), jnp.float32),
              jax.ShapeDtypeStruct((NPAD, D), jnp.float32)),
    mesh=_MESH,
    scratch_types=[
        pltpu.VMEM((B, CHUNK), jnp.int32),
        pltpu.VMEM((B, CHUNK), jnp.int32),
        pltpu.VMEM((B, CHUNK), jnp.int32),
        pltpu.VMEM((B, CHUNK), jnp.int32),
        pltpu.VMEM((CHUNK, D), jnp.float32),
        pltpu.VMEM((CHUNK, D), jnp.float32),
        pltpu.VMEM_SHARED((NPAD, D), jnp.float32),
        pltpu.SemaphoreType.DMA,
        pltpu.SemaphoreType.DMA,
        pltpu.SemaphoreType.DMA,
        pltpu.SemaphoreType.DMA,
    ],
    compiler_params=_SC_PARAMS,
)


def _out_body(a0_ref, a1_ref, degp_ref, w_ref, b_ref, pa_ref, o_ref):
    deg = jnp.sum(degp_ref[...], axis=0)
    norm = jnp.where(deg > 0, lax.rsqrt(deg), 0.0)
    rst = (a0_ref[0:N_NODES] + a1_ref[0:N_NODES]) * norm[0:N_NODES, None]
    o = jnp.dot(rst, w_ref[...], preferred_element_type=jnp.float32) + b_ref[...]
    a = pa_ref[0, 0]
    o_ref[...] = jnp.where(o >= 0, o, a * o)


def _finish(a0, a1, degi_p, W, b2, pa2):
    return pl.pallas_call(
        _out_body,
        in_specs=[
            pl.BlockSpec(memory_space=pltpu.VMEM),
            pl.BlockSpec(memory_space=pltpu.VMEM),
            pl.BlockSpec(memory_space=pltpu.VMEM),
            pl.BlockSpec(memory_space=pltpu.VMEM),
            pl.BlockSpec(memory_space=pltpu.VMEM),
            pl.BlockSpec(memory_space=pltpu.SMEM),
        ],
        out_shape=jax.ShapeDtypeStruct((N_NODES, D), jnp.float32),
    )(a0, a1, degi_p, W, b2, pa2)


def kernel(feats, edge_index, W, b, prelu_a):
    src = edge_index[0].astype(jnp.int32)
    dst = edge_index[1].astype(jnp.int32)
    cyc = jnp.arange(EPAD - N_EDGES, dtype=jnp.int32) % (NPAD - N_NODES)
    pad = N_NODES + cyc
    pad0 = cyc
    src_p = jnp.concatenate([src, pad]).reshape(NW, E_PER_W)
    dst_p = jnp.concatenate([dst, pad]).reshape(NW, E_PER_W)
    src_c = jnp.concatenate([src, pad0]).reshape(NW, NB, B, CHUNK)
    dst_c = dst_p.reshape(NW, NB, B, CHUNK)

    dego_p, degi_p = _deg_kernel(src_p, dst_p)
    h = _scale(feats, dego_p)
    a0, a1 = _agg_kernel(h, src_c, dst_c)
    out = _finish(a0, a1, degi_p, W,
                  b.reshape(1, D), prelu_a.reshape(1, 1))
    return out
